# baseline (speedup 1.0000x reference)
import functools

import jax
import jax.numpy as jnp
from jax import lax
from jax.experimental import pallas as pl
from jax.experimental.pallas import tpu as pltpu
from jax.experimental.pallas import tpu_sc as plsc

B = 1024
COUNT_DIM = 8
NUM_PROPS = 1032
FP_DIM = NUM_PROPS - COUNT_DIM
N_EMBD = 64
K = NUM_PROPS // 2
NC, NS = 2, 16
NW = NC * NS
BPW = B // NW
PAIRS = BPW * K
S = 192
NSTAGES = PAIRS // S

STAGE_CHUNKS = ((0, 120), (120, 72))

_mesh = plsc.VectorSubcoreMesh(core_axis_name="c", subcore_axis_name="s")


@functools.partial(
    pl.kernel,
    mesh=_mesh,
    out_type=jax.ShapeDtypeStruct((B * K, 2 * N_EMBD), jnp.float32),
    scratch_types=[
        pltpu.VMEM((PAIRS,), jnp.int32),
        pltpu.VMEM((S,), jnp.int32),
        pltpu.VMEM((S,), jnp.int32),
        pltpu.VMEM((S, 2 * N_EMBD), jnp.float32),
        pltpu.VMEM((S, 2 * N_EMBD), jnp.float32),
        pltpu.VMEM_SHARED((4 * K, 2 * N_EMBD), jnp.float32),
        pltpu.SemaphoreType.DMA,
        pltpu.SemaphoreType.DMA,
        pltpu.SemaphoreType.DMA,
    ],
)
def _sc_embed(q_hbm, table_hbm, out_hbm, q_v, idx_a, idx_b, buf_a, buf_b,
              table_s, sem_g, sem_a, sem_b):
    sid = lax.axis_index("s")
    wid = lax.axis_index("c") * NS + sid
    slot0 = wid * PAIRS

    @pl.when(sid == 0)
    def _():
        pltpu.sync_copy(table_hbm, table_s)

    pltpu.sync_copy(q_hbm.at[pl.ds(slot0, PAIRS)], q_v)
    plsc.subcore_barrier()

    def build_idx(idx_v, stage, k_vec):
        def per_chunk(c, k):
            p = stage * S + c * 16
            idx_v[pl.ds(c * 16, 16)] = q_v[pl.ds(p, 16)] * K + k
            k = k + 16
            return jnp.where(k >= K, k - K, k)

        return lax.fori_loop(0, S // 16, per_chunk, k_vec)

    def fire_gathers(idx_v, buf_v):
        return [
            pltpu.async_copy(
                table_s.at[idx_v.at[pl.ds(off, n)]],
                buf_v.at[pl.ds(off, n)],
                sem_g,
            )
            for off, n in STAGE_CHUNKS
        ]

    def drain(buf_v, sem):
        pltpu.make_async_copy(out_hbm.at[pl.ds(0, S)], buf_v, sem).wait()

    def per_pair(j, k_vec):
        sa = 2 * j

        @pl.when(j > 0)
        def _():
            drain(buf_a, sem_a)
        k_vec = build_idx(idx_a, sa, k_vec)
        for cp in fire_gathers(idx_a, buf_a):
            cp.wait()
        pltpu.async_copy(buf_a, out_hbm.at[pl.ds(slot0 + sa * S, S)], sem_a)

        @pl.when(j > 0)
        def _():
            drain(buf_b, sem_b)
        k_vec = build_idx(idx_b, sa + 1, k_vec)
        for cp in fire_gathers(idx_b, buf_b):
            cp.wait()
        pltpu.async_copy(buf_b, out_hbm.at[pl.ds(slot0 + (sa + 1) * S, S)],
                         sem_b)
        return k_vec

    lax.fori_loop(0, NSTAGES // 2, per_pair,
                  lax.broadcasted_iota(jnp.int32, (16,), 0))

    drain(buf_a, sem_a)
    drain(buf_b, sem_b)


def _build_table(type_emb, count_val, count_bit, fp_pair, fp_bit, fp_val):
    base_c = count_val[0] + count_bit + type_emb[0]
    base_f = (fp_val[0]
              + jnp.repeat(fp_pair, 2, axis=0)
              + jnp.tile(fp_bit, (FP_DIM // 2, 1))
              + type_emb[1])
    base = jnp.concatenate([base_c, base_f], axis=0)
    delta_c = jnp.broadcast_to(count_val[1] - count_val[0],
                               (COUNT_DIM, N_EMBD))
    delta_f = jnp.broadcast_to(fp_val[1] - fp_val[0], (FP_DIM, N_EMBD))
    delta = jnp.concatenate([delta_c, delta_f], axis=0)
    full = jnp.stack([base, base + delta])
    even = full[:, 0::2, :]
    odd = full[:, 1::2, :]
    paired = jnp.concatenate([
        jnp.broadcast_to(even[:, None], (2, 2, K, N_EMBD)),
        jnp.broadcast_to(odd[None, :], (2, 2, K, N_EMBD)),
    ], axis=-1)
    return paired.reshape(4 * K, 2 * N_EMBD)


def kernel(prop, type_emb, count_val, count_bit, fp_pair, fp_bit, fp_val):
    table = _build_table(type_emb, count_val, count_bit, fp_pair, fp_bit,
                         fp_val)
    q = 2 * prop[:, 0::2] + prop[:, 1::2]
    out = _sc_embed(q.reshape(-1), table)
    return out.reshape(B, NUM_PROPS, N_EMBD)
